# Initial kernel scaffold; baseline (speedup 1.0000x reference)
import jax
import jax.numpy as jnp
from jax.experimental import pallas as pl


def kernel(inputs, weight):
    raise NotImplementedError("write your pallas kernel here")


# speedup vs baseline: 12.9955x; 12.9955x over previous
import functools

import jax
import jax.numpy as jnp
from jax import lax
from jax.experimental import pallas as pl
from jax.experimental.pallas import tpu as pltpu
from jax.experimental.pallas import tpu_sc as plsc

NUM_CODES = 8192
DIM = 32
N_TOKENS = 16384
TOK_TILE = 256
CODE_CHUNK = 2048
N_CHUNKS = NUM_CODES // CODE_CHUNK
GRID = N_TOKENS // TOK_TILE

NUM_WORKERS = 32
ROWS_PER_WORKER = N_TOKENS // NUM_WORKERS
IDX_MINOR = 128
SUB_GATHERS = ROWS_PER_WORKER // IDX_MINOR


def _argmin_body(flat_ref, wt_ref, fsq_ref, wsq_ref, idx_ref, loss_ref):
    i = pl.program_id(0)
    f = flat_ref[...]
    fsq = fsq_ref[...]
    best_val = None
    best_idx = None
    for c in range(N_CHUNKS):
        sl = slice(c * CODE_CHUNK, (c + 1) * CODE_CHUNK)
        w_c = wt_ref[:, sl]
        wsq_c = wsq_ref[:, sl]
        mm = lax.dot_general(f, w_c, (((1,), (0,)), ((), ())),
                             preferred_element_type=jnp.float32)
        d = (fsq + wsq_c) - 2.0 * mm
        m = jnp.min(d, axis=1, keepdims=True)
        io = lax.broadcasted_iota(jnp.int32, (TOK_TILE, CODE_CHUNK), 1)
        io = io.astype(jnp.float32) + float(c * CODE_CHUNK)
        ci = jnp.min(jnp.where(d == m, io, 3.0e8), axis=1, keepdims=True)
        if best_val is None:
            best_val, best_idx = m, ci
        else:
            upd = m < best_val
            best_val = jnp.where(upd, m, best_val)
            best_idx = jnp.where(upd, ci, best_idx)
    idx_ref[...] = best_idx.astype(jnp.int32)
    part = jnp.sum(best_val)

    @pl.when(i == 0)
    def _():
        loss_ref[...] = jnp.reshape(part, (1, 1))

    @pl.when(i > 0)
    def _():
        loss_ref[...] = loss_ref[...] + part

    @pl.when(i == GRID - 1)
    def _():
        loss_ref[...] = loss_ref[...] * (1.25 / float(N_TOKENS * DIM))


_argmin_call = pl.pallas_call(
    _argmin_body,
    grid=(GRID,),
    in_specs=[
        pl.BlockSpec((TOK_TILE, DIM), lambda i: (i, 0)),
        pl.BlockSpec((DIM, NUM_CODES), lambda i: (0, 0)),
        pl.BlockSpec((TOK_TILE, 1), lambda i: (i, 0)),
        pl.BlockSpec((1, NUM_CODES), lambda i: (0, 0)),
    ],
    out_specs=[
        pl.BlockSpec((TOK_TILE, 1), lambda i: (i, 0)),
        pl.BlockSpec((1, 1), lambda i: (0, 0)),
    ],
    out_shape=[
        jax.ShapeDtypeStruct((N_TOKENS, 1), jnp.int32),
        jax.ShapeDtypeStruct((1, 1), jnp.float32),
    ],
)


@functools.lru_cache(maxsize=1)
def _make_gather_rows():
    mesh = plsc.VectorSubcoreMesh(core_axis_name="c", subcore_axis_name="s")

    @functools.partial(
        pl.kernel,
        out_type=jax.ShapeDtypeStruct((N_TOKENS, DIM), jnp.float32),
        mesh=mesh,
        compiler_params=pltpu.CompilerParams(use_tc_tiling_on_sc=False),
        scratch_types=[
            pltpu.VMEM((SUB_GATHERS, IDX_MINOR), jnp.int32),
            pltpu.VMEM((ROWS_PER_WORKER, DIM), jnp.float32),
            pltpu.SemaphoreType.DMA,
        ],
    )
    def _gather_rows(idx_hbm, table_hbm, out_hbm, idx_v, rows_v, sem):
        wid = lax.axis_index("s") * 2 + lax.axis_index("c")
        base = wid * ROWS_PER_WORKER
        pltpu.sync_copy(idx_hbm.at[pl.ds(wid * SUB_GATHERS, SUB_GATHERS)],
                        idx_v)
        copies = []
        for j in range(SUB_GATHERS):
            copies.append(pltpu.async_copy(
                table_hbm.at[idx_v.at[j]],
                rows_v.at[pl.ds(j * IDX_MINOR, IDX_MINOR)],
                sem))
        for cp in copies:
            cp.wait()
        pltpu.sync_copy(rows_v, out_hbm.at[pl.ds(base, ROWS_PER_WORKER)])

    return _gather_rows


def kernel(inputs, weight):
    flat = inputs.reshape(-1, DIM)
    fsq = jnp.sum(flat ** 2, axis=1, keepdims=True)
    wsq = jnp.sum(weight ** 2, axis=1).reshape(1, NUM_CODES)
    wt = weight.T
    idx, loss = _argmin_call(flat, wt, fsq, wsq)
    quantized = _make_gather_rows()(
        idx.reshape(NUM_WORKERS * SUB_GATHERS, IDX_MINOR), weight)
    return (quantized, loss[0, 0], idx)
